# Initial kernel scaffold; baseline (speedup 1.0000x reference)
import jax
import jax.numpy as jnp
from jax.experimental import pallas as pl


def kernel(t, theta):
    raise NotImplementedError("write your pallas kernel here")


# speedup vs baseline: 3443.2726x; 3443.2726x over previous
import functools

import jax
import jax.numpy as jnp
from jax import lax
from jax.experimental import pallas as pl
from jax.experimental.pallas import tpu as pltpu
from jax.experimental.pallas import tpu_sc as plsc

K = 64
G0 = 1e-4
T_END = 80.0
EPS = 1e-18


def _tables(theta):
    log_g0 = jnp.log(jnp.float32(G0))
    log_T = jnp.log(jnp.float32(T_END))
    log_gap = log_T - log_g0
    s_knots = jnp.linspace(0.0, 1.0, K, dtype=jnp.float32)
    w = jax.nn.softmax(theta, axis=0)
    y = jnp.cumsum(jnp.concatenate([jnp.zeros((1,), theta.dtype), w]))
    x0, x1 = s_knots[:-1], s_knots[1:]
    y0, y1 = y[:-1], y[1:]
    a = (y1 - y0) / (x1 - x0 + EPS)
    b = y0 - x0 * a
    A = log_gap * a
    B = log_g0 + log_gap * b
    Atab = jnp.concatenate([A[:1], A])
    Btab = jnp.concatenate([B[:1], B])
    return Atab, Btab


def _make_body(nw, lanes, per_w, ch, nch):
    def body(t_hbm, A_hbm, B_hbm, gw_hbm, gd_hbm,
             tabA, tabB, tin0, tin1, gw0, gw1, gd0, gd1,
             si0, si1, sgw0, sgw1, sgd0, sgd1):
        cid = lax.axis_index("c")
        sid = lax.axis_index("s")
        wid = sid * 2 + cid
        base = wid * per_w
        pltpu.sync_copy(A_hbm, tabA)
        pltpu.sync_copy(B_hbm, tabB)

        tin = (tin0, tin1)
        gwv = (gw0, gw1)
        gdv = (gd0, gd1)
        si = (si0, si1)
        sgw = (sgw0, sgw1)
        sgd = (sgd0, sgd1)
        in_cp = [None, None]
        out_cp = [None, None]

        def start_in(c):
            b = c % 2
            in_cp[b] = pltpu.async_copy(
                t_hbm.at[pl.ds(base + c * ch, ch)], tin[b], si[b])

        start_in(0)
        for c in range(nch):
            b = c % 2
            in_cp[b].wait()
            if c + 1 < nch:
                start_in(c + 1)
            if out_cp[b] is not None:
                gwc, gdc = out_cp[b]
                gwc.wait()
                gdc.wait()

            def compute(k, _, b=b):
                tv = tin[b][pl.ds(k * lanes, lanes)]
                j = jnp.clip((tv * jnp.float32(63.0)).astype(jnp.int32) + 1,
                             1, 63)
                Aj = plsc.load_gather(tabA, [j])
                Bj = plsc.load_gather(tabB, [j])
                gw = jnp.exp(Bj + Aj * tv)
                gwv[b][pl.ds(k * lanes, lanes)] = gw
                gdv[b][pl.ds(k * lanes, lanes)] = gw * Aj
                return _

            lax.fori_loop(0, ch // lanes, compute, None)
            out_cp[b] = (
                pltpu.async_copy(gwv[b], gw_hbm.at[pl.ds(base + c * ch, ch)],
                                 sgw[b]),
                pltpu.async_copy(gdv[b], gd_hbm.at[pl.ds(base + c * ch, ch)],
                                 sgd[b]),
            )
        for b in range(2):
            if out_cp[b] is not None:
                gwc, gdc = out_cp[b]
                gwc.wait()
                gdc.wait()

    return body


@jax.jit
def kernel(t, theta):
    n = t.shape[0]
    info = plsc.get_sparse_core_info()
    nc, ns, lanes = info.num_cores, info.num_subcores, info.num_lanes
    nw = nc * ns
    per_w = n // nw
    ch = 16384
    nch = per_w // ch

    Atab, Btab = _tables(theta)

    mesh = plsc.VectorSubcoreMesh(core_axis_name="c", subcore_axis_name="s")
    f32 = jnp.float32
    fn = pl.kernel(
        _make_body(nw, lanes, per_w, ch, nch),
        out_type=(jax.ShapeDtypeStruct((n,), f32),
                  jax.ShapeDtypeStruct((n,), f32)),
        mesh=mesh,
        compiler_params=pltpu.CompilerParams(needs_layout_passes=False),
        scratch_types=(
            [pltpu.VMEM((K,), f32)] * 2
            + [pltpu.VMEM((ch,), f32)] * 6
            + [pltpu.SemaphoreType.DMA] * 6
        ),
    )
    gw, gwd = fn(t, Atab, Btab)
    return (gw, gwd)


# speedup vs baseline: 13563.7806x; 3.9392x over previous
import functools

import jax
import jax.numpy as jnp
from jax import lax
from jax.experimental import pallas as pl
from jax.experimental.pallas import tpu as pltpu
from jax.experimental.pallas import tpu_sc as plsc

K = 64
G0 = 1e-4
T_END = 80.0
EPS = 1e-18


def _tables(theta):
    log_g0 = jnp.log(jnp.float32(G0))
    log_T = jnp.log(jnp.float32(T_END))
    log_gap = log_T - log_g0
    s_knots = jnp.linspace(0.0, 1.0, K, dtype=jnp.float32)
    w = jax.nn.softmax(theta, axis=0)
    y = jnp.cumsum(jnp.concatenate([jnp.zeros((1,), theta.dtype), w]))
    x0, x1 = s_knots[:-1], s_knots[1:]
    y0, y1 = y[:-1], y[1:]
    a = (y1 - y0) / (x1 - x0 + EPS)
    b = y0 - x0 * a
    A = log_gap * a
    B = log_g0 + log_gap * b
    Atab = jnp.concatenate([A[:1], A])
    Btab = jnp.concatenate([B[:1], B])
    return Atab, Btab


def _make_body(nw, lanes, per_w, ch, nch):
    def body(t_hbm, A_hbm, B_hbm, gw_hbm, gd_hbm,
             tabA, tabB, tin0, tin1, gw0, gw1, gd0, gd1,
             si0, si1, sgw0, sgw1, sgd0, sgd1):
        cid = lax.axis_index("c")
        sid = lax.axis_index("s")
        wid = sid * 2 + cid
        base = wid * per_w
        pltpu.sync_copy(A_hbm, tabA)
        pltpu.sync_copy(B_hbm, tabB)

        tin = (tin0, tin1)
        gwv = (gw0, gw1)
        gdv = (gd0, gd1)
        si = (si0, si1)
        sgw = (sgw0, sgw1)
        sgd = (sgd0, sgd1)
        in_cp = [None, None]
        out_cp = [None, None]

        def start_in(c):
            b = c % 2
            in_cp[b] = pltpu.async_copy(
                t_hbm.at[pl.ds(base + c * ch, ch)], tin[b], si[b])

        start_in(0)
        for c in range(nch):
            b = c % 2
            in_cp[b].wait()
            if c + 1 < nch:
                start_in(c + 1)
            if out_cp[b] is not None:
                gwc, gdc = out_cp[b]
                gwc.wait()
                gdc.wait()

            @plsc.parallel_loop(0, ch // lanes, step=1, unroll=8)
            def _(k, b=b):
                tv = tin[b][pl.ds(k * lanes, lanes)]
                j = jnp.clip((tv * jnp.float32(63.0)).astype(jnp.int32) + 1,
                             1, 63)
                Aj = plsc.load_gather(tabA, [j])
                Bj = plsc.load_gather(tabB, [j])
                gw = jnp.exp(Bj + Aj * tv)
                gwv[b][pl.ds(k * lanes, lanes)] = gw
                gdv[b][pl.ds(k * lanes, lanes)] = gw * Aj
            out_cp[b] = (
                pltpu.async_copy(gwv[b], gw_hbm.at[pl.ds(base + c * ch, ch)],
                                 sgw[b]),
                pltpu.async_copy(gdv[b], gd_hbm.at[pl.ds(base + c * ch, ch)],
                                 sgd[b]),
            )
        for b in range(2):
            if out_cp[b] is not None:
                gwc, gdc = out_cp[b]
                gwc.wait()
                gdc.wait()

    return body


@jax.jit
def kernel(t, theta):
    n = t.shape[0]
    info = plsc.get_sparse_core_info()
    nc, ns, lanes = info.num_cores, info.num_subcores, info.num_lanes
    nw = nc * ns
    per_w = n // nw
    ch = 16384
    nch = per_w // ch

    Atab, Btab = _tables(theta)

    mesh = plsc.VectorSubcoreMesh(core_axis_name="c", subcore_axis_name="s")
    f32 = jnp.float32
    fn = pl.kernel(
        _make_body(nw, lanes, per_w, ch, nch),
        out_type=(jax.ShapeDtypeStruct((n,), f32),
                  jax.ShapeDtypeStruct((n,), f32)),
        mesh=mesh,
        compiler_params=pltpu.CompilerParams(needs_layout_passes=False),
        scratch_types=(
            [pltpu.VMEM((K,), f32)] * 2
            + [pltpu.VMEM((ch,), f32)] * 6
            + [pltpu.SemaphoreType.DMA] * 6
        ),
    )
    gw, gwd = fn(t, Atab, Btab)
    return (gw, gwd)
